# Initial kernel scaffold; baseline (speedup 1.0000x reference)
import jax
import jax.numpy as jnp
from jax.experimental import pallas as pl


def kernel(x, edge_index, batch, bu1_W, bu1_b, td1_W, td1_b, root_W, root_b, bu2_W, bu2_b, td2_W, td2_b, lin_W, lin_b):
    raise NotImplementedError("write your pallas kernel here")


# speedup vs baseline: 7.7600x; 7.7600x over previous
import functools

import jax
import jax.numpy as jnp
from jax import lax
from jax.experimental import pallas as pl
from jax.experimental.pallas import tpu as pltpu
from jax.experimental.pallas import tpu_sc as plsc

N = 10000
E = 320000
F = 128
G = 64
C = 4
NB = 400
NBLK = N // NB
NSC = 2
NTEC = 16
EPT = E // (NSC * NTEC) * NSC
EPT = E // NTEC
ECH = 128
NFULL = EPT // ECH
EREM = EPT - NFULL * ECH
NPAD = 10240
RPT = N // NTEC


def _sc_mesh():
    return plsc.VectorSubcoreMesh(core_axis_name="c", subcore_axis_name="s",
                                  num_cores=NSC, num_subcores=NTEC)


def _writeout_split(copy_fn):
    sid = lax.axis_index("s")

    @pl.when(sid < NTEC - 1)
    def _():
        copy_fn(sid * 640, 640)

    @pl.when(sid == NTEC - 1)
    def _():
        copy_fn((NTEC - 1) * 640, N - (NTEC - 1) * 640)


def _root_kernel(first, x, roots, idxv, rows, sem):
    cid = lax.axis_index("c")
    sid = lax.axis_index("s")

    @pl.when(jnp.logical_and(cid == 0, sid == 0))
    def _():
        pltpu.sync_copy(first, idxv)
        pltpu.async_copy(x.at[idxv], rows, sem).wait()
        pltpu.sync_copy(rows, roots)


def _sc_root_gather(first, x):
    k = pl.kernel(
        _root_kernel,
        out_type=jax.ShapeDtypeStruct((G, F), jnp.float32),
        mesh=_sc_mesh(),
        scratch_types=[
            pltpu.VMEM((G,), jnp.int32),
            pltpu.VMEM((G, F), jnp.float32),
            pltpu.SemaphoreType.DMA,
        ],
    )
    return k(first, x)


def _agg_kernel(p, ei0, ei1, zrows,
                out, acc, rows, rows2, sidx, didx, sidx2, didx2, sem):
    cid = lax.axis_index("c")
    sid = lax.axis_index("s")
    pltpu.sync_copy(zrows, acc.at[pl.ds(sid * 640, 640)])
    plsc.subcore_barrier()

    for d in (0, 1):
        src_e = ei0 if d == 0 else ei1
        dst_e = ei1 if d == 0 else ei0

        @pl.when(cid == d)
        def _():
            base = sid * EPT

            @pl.loop(0, NFULL)
            def _(i):
                off = base + i * ECH
                pltpu.sync_copy(src_e.at[pl.ds(off, ECH)], sidx)
                pltpu.sync_copy(dst_e.at[pl.ds(off, ECH)], didx)
                pltpu.async_copy(p.at[d].at[sidx], rows, sem).wait()
                pltpu.sync_copy(rows, acc.at[didx], add=True)

            off = base + NFULL * ECH
            pltpu.sync_copy(src_e.at[pl.ds(off, EREM)], sidx2)
            pltpu.sync_copy(dst_e.at[pl.ds(off, EREM)], didx2)
            pltpu.async_copy(p.at[d].at[sidx2], rows2, sem).wait()
            pltpu.sync_copy(rows2, acc.at[didx2], add=True)

    plsc.subcore_barrier()

    for d in (0, 1):
        @pl.when(cid == d)
        def _():
            _writeout_split(
                lambda r0, n: pltpu.sync_copy(acc.at[pl.ds(r0, n)],
                                              out.at[d].at[pl.ds(r0, n)]))


def _sc_edge_agg(p, ei0, ei1, zrows):
    k = pl.kernel(
        _agg_kernel,
        out_type=jax.ShapeDtypeStruct((2, N, F), jnp.float32),
        mesh=_sc_mesh(),
        scratch_types=[
            pltpu.VMEM_SHARED((NPAD, F), jnp.float32),
            pltpu.VMEM((ECH, F), jnp.float32),
            pltpu.VMEM((EREM, F), jnp.float32),
            pltpu.VMEM((ECH,), jnp.int32),
            pltpu.VMEM((ECH,), jnp.int32),
            pltpu.VMEM((EREM,), jnp.int32),
            pltpu.VMEM((EREM,), jnp.int32),
            pltpu.SemaphoreType.DMA,
        ],
    )
    return k(p, ei0, ei1, zrows)


EB = 512
EBLK = E // EB


def _hist_kernel(e0, e1, d0t, d1t):
    i = pl.program_id(0)

    @pl.when(i == 0)
    def _():
        d0t[...] = jnp.zeros_like(d0t)
        d1t[...] = jnp.zeros_like(d1t)

    iq = lax.broadcasted_iota(jnp.int32, (EB, NBLK), 1)
    ir = lax.broadcasted_iota(jnp.int32, (EB, NB), 1)
    for dst_ref, out in ((e1, d0t), (e0, d1t)):
        dst = dst_ref[0]
        q = dst // NB
        r = dst - q * NB
        ohq = (q == iq).astype(jnp.bfloat16)
        ohr = (r == ir).astype(jnp.bfloat16)
        out[...] += lax.dot_general(ohr, ohq, (((0,), (0,)), ((), ())),
                                    preferred_element_type=jnp.float32)


def _tc_deghist(e0c, e1c):
    return pl.pallas_call(
        _hist_kernel,
        grid=(EBLK,),
        in_specs=[
            pl.BlockSpec((1, EB, 1), lambda i: (i, 0, 0)),
            pl.BlockSpec((1, EB, 1), lambda i: (i, 0, 0)),
        ],
        out_specs=[
            pl.BlockSpec((NB, NBLK), lambda i: (0, 0)),
            pl.BlockSpec((NB, NBLK), lambda i: (0, 0)),
        ],
        out_shape=[
            jax.ShapeDtypeStruct((NB, NBLK), jnp.float32),
            jax.ShapeDtypeStruct((NB, NBLK), jnp.float32),
        ],
    )(e0c, e1c)


def _mm_kernel(x, w, o):
    o[...] = jnp.dot(x[...], w[...], preferred_element_type=jnp.float32)


def _tc_h(x, wcat):
    return pl.pallas_call(
        _mm_kernel,
        grid=(NBLK,),
        in_specs=[
            pl.BlockSpec((NB, F), lambda i: (i, 0)),
            pl.BlockSpec((F, 2 * F), lambda i: (0, 0)),
        ],
        out_specs=pl.BlockSpec((NB, 2 * F), lambda i: (i, 0)),
        out_shape=jax.ShapeDtypeStruct((N, 2 * F), jnp.float32),
    )(x, wcat)


def _scale_kernel(h, degt, p, dis):
    i = pl.program_id(1)
    lane = lax.broadcasted_iota(jnp.int32, (NB, NBLK), 1)
    deg = jnp.sum(degt[0] * (lane == i).astype(jnp.float32),
                  axis=1, keepdims=True)
    d = jax.lax.rsqrt(deg + 1.0)
    p[0] = h[...] * d
    dis[0] = d


def _tc_scale(h, degt):
    return pl.pallas_call(
        _scale_kernel,
        grid=(2, NBLK),
        in_specs=[
            pl.BlockSpec((NB, F), lambda d, i: (i, d)),
            pl.BlockSpec((1, NB, NBLK), lambda d, i: (d, 0, 0)),
        ],
        out_specs=[
            pl.BlockSpec((1, NB, F), lambda d, i: (d, i, 0)),
            pl.BlockSpec((1, NB, 1), lambda d, i: (d, i, 0)),
        ],
        out_shape=[
            jax.ShapeDtypeStruct((2, N, F), jnp.float32),
            jax.ShapeDtypeStruct((2, N, 1), jnp.float32),
        ],
    )(h, degt)


def _first_kernel(bc, first, cnt, cacc):
    i = pl.program_id(0)

    @pl.when(i == 0)
    def _():
        cacc[...] = jnp.zeros_like(cacc)

    gidx = lax.broadcasted_iota(jnp.int32, (NB, G), 1)
    oh = (bc[...] == gidx).astype(jnp.float32)
    cacc[...] += jnp.sum(oh, axis=0, keepdims=True)

    @pl.when(i == NBLK - 1)
    def _():
        c = cacc[...]
        gj = lax.broadcasted_iota(jnp.int32, (G, G), 0)
        gg = lax.broadcasted_iota(jnp.int32, (G, G), 1)
        lt = (gj < gg).astype(jnp.float32)
        f = jnp.dot(c, lt, preferred_element_type=jnp.float32)
        first[...] = jnp.clip(f.astype(jnp.int32), 0, N - 1)
        cnt[...] = c


def _tc_first(batch_c):
    return pl.pallas_call(
        _first_kernel,
        grid=(NBLK,),
        in_specs=[pl.BlockSpec((NB, 1), lambda i: (i, 0))],
        out_specs=[
            pl.BlockSpec((1, G), lambda i: (0, 0)),
            pl.BlockSpec((1, G), lambda i: (0, 0)),
        ],
        out_shape=[
            jax.ShapeDtypeStruct((1, G), jnp.int32),
            jax.ShapeDtypeStruct((1, G), jnp.float32),
        ],
        scratch_shapes=[pltpu.VMEM((1, G), jnp.float32)],
    )(batch_c)


def _q_kernel(roots, wbot, q):
    q[...] = jnp.dot(jax.nn.relu(roots[...]), wbot[...],
                     preferred_element_type=jnp.float32)


def _tc_q(roots, wbot_cat):
    return pl.pallas_call(
        _q_kernel,
        out_shape=jax.ShapeDtypeStruct((G, 2 * F), jnp.float32),
    )(roots, wbot_cat)


def _layer2_kernel(pp, agg, dis, bc, q, wtop, b1, b2, p2, basev):
    b = bc[...]
    gidx = lax.broadcasted_iota(jnp.int32, (NB, G), 1)
    oh = (b == gidx).astype(jnp.float32)
    for d in (0, 1):
        dd = dis[d]
        conv1 = dd * (agg[d] + pp[d]) + b1[d]
        t = jax.nn.relu(conv1)
        h2 = (jnp.dot(t, wtop[d], preferred_element_type=jnp.float32)
              + jnp.dot(oh, q[:, d * F:(d + 1) * F],
                        preferred_element_type=jnp.float32))
        p2d = dd * h2
        p2[d] = p2d
        basev[d] = dd * p2d + b2[d]


def _tc_layer2(p, agg, dis, batch_c, q, wtop, b1, b2):
    return pl.pallas_call(
        _layer2_kernel,
        grid=(NBLK,),
        in_specs=[
            pl.BlockSpec((2, NB, F), lambda i: (0, i, 0)),
            pl.BlockSpec((2, NB, F), lambda i: (0, i, 0)),
            pl.BlockSpec((2, NB, 1), lambda i: (0, i, 0)),
            pl.BlockSpec((NB, 1), lambda i: (i, 0)),
            pl.BlockSpec((G, 2 * F), lambda i: (0, 0)),
            pl.BlockSpec((2, F, F), lambda i: (0, 0, 0)),
            pl.BlockSpec((2, F), lambda i: (0, 0)),
            pl.BlockSpec((2, F), lambda i: (0, 0)),
        ],
        out_specs=[
            pl.BlockSpec((2, NB, F), lambda i: (0, i, 0)),
            pl.BlockSpec((2, NB, F), lambda i: (0, i, 0)),
        ],
        out_shape=[
            jax.ShapeDtypeStruct((2, N, F), jnp.float32),
            jax.ShapeDtypeStruct((2, N, F), jnp.float32),
        ],
    )(p, agg, dis, batch_c, q, wtop, b1, b2)


def _final_kernel(agg2, basev, dis, bc, cnt, lw, lb, out, acc):
    i = pl.program_id(0)

    @pl.when(i == 0)
    def _():
        acc[...] = jnp.zeros_like(acc)

    b = bc[...]
    gidx = lax.broadcasted_iota(jnp.int32, (NB, G), 1)
    oh = (b == gidx).astype(jnp.float32)
    for d in (0, 1):
        h = jax.nn.relu(dis[d] * agg2[d] + basev[d])
        contrib = lax.dot_general(oh, h, (((0,), (0,)), ((), ())),
                                  preferred_element_type=jnp.float32)
        acc[:, d * F:(d + 1) * F] += contrib

    @pl.when(i == NBLK - 1)
    def _():
        gr = lax.broadcasted_iota(jnp.int32, (G, G), 0)
        gc = lax.broadcasted_iota(jnp.int32, (G, G), 1)
        dm = (gr == gc).astype(jnp.float32) / jnp.maximum(cnt[...], 1.0)
        mean = jnp.dot(dm, acc[...], preferred_element_type=jnp.float32)
        logits = jnp.dot(mean, lw[...],
                         preferred_element_type=jnp.float32) + lb[...]
        m = jnp.max(logits, axis=-1, keepdims=True)
        z = logits - m
        out[...] = z - jnp.log(jnp.sum(jnp.exp(z), axis=-1, keepdims=True))


def _tc_final(agg2, basev, dis, batch_c, cnt, lw, lb):
    return pl.pallas_call(
        _final_kernel,
        grid=(NBLK,),
        in_specs=[
            pl.BlockSpec((2, NB, F), lambda i: (0, i, 0)),
            pl.BlockSpec((2, NB, F), lambda i: (0, i, 0)),
            pl.BlockSpec((2, NB, 1), lambda i: (0, i, 0)),
            pl.BlockSpec((NB, 1), lambda i: (i, 0)),
            pl.BlockSpec((1, G), lambda i: (0, 0)),
            pl.BlockSpec((2 * F, C), lambda i: (0, 0)),
            pl.BlockSpec((1, C), lambda i: (0, 0)),
        ],
        out_specs=pl.BlockSpec((G, C), lambda i: (0, 0)),
        out_shape=jax.ShapeDtypeStruct((G, C), jnp.float32),
        scratch_shapes=[pltpu.VMEM((G, 2 * F), jnp.float32)],
    )(agg2, basev, dis, batch_c, cnt, lw, lb)


def kernel(x, edge_index, batch, bu1_W, bu1_b, td1_W, td1_b, root_W, root_b,
           bu2_W, bu2_b, td2_W, td2_b, lin_W, lin_b):
    ei = edge_index.astype(jnp.int32)
    ei0, ei1 = ei[0], ei[1]
    batch_c = batch.astype(jnp.int32).reshape(N, 1)
    zrows = jnp.zeros((640, F), jnp.float32)

    wcat = jnp.concatenate([bu1_W, td1_W], axis=1)
    wtop = jnp.stack([bu2_W[:F], td2_W[:F]])
    wbot_cat = jnp.concatenate([bu2_W[F:], td2_W[F:]], axis=1)
    b1 = jnp.stack([bu1_b, td1_b])
    b2 = jnp.stack([bu2_b, td2_b])

    e0c = ei0.reshape(EBLK, EB, 1)
    e1c = ei1.reshape(EBLK, EB, 1)
    d0t, d1t = _tc_deghist(e0c, e1c)
    h = _tc_h(x, wcat)

    p, dis = _tc_scale(h, jnp.stack([d0t, d1t]))
    first, cnt = _tc_first(batch_c)

    roots = _sc_root_gather(first.reshape(G), x)
    q = _tc_q(roots, wbot_cat)

    agg = _sc_edge_agg(p, ei0, ei1, zrows)
    p2, basev = _tc_layer2(p, agg, dis, batch_c, q, wtop, b1, b2)
    agg2 = _sc_edge_agg(p2, ei0, ei1, zrows)
    return _tc_final(agg2, basev, dis, batch_c, cnt,
                     lin_W, lin_b.reshape(1, C))


# speedup vs baseline: 9.1963x; 1.1851x over previous
import functools

import jax
import jax.numpy as jnp
from jax import lax
from jax.experimental import pallas as pl
from jax.experimental.pallas import tpu as pltpu
from jax.experimental.pallas import tpu_sc as plsc

N = 10000
E = 320000
F = 128
G = 64
C = 4
NB = 400
NBLK = N // NB
NSC = 2
NTEC = 16
ECH = 128
NCH = 157
EPAD = NTEC * NCH * ECH
NPAD = 10240


def _sc_mesh():
    return plsc.VectorSubcoreMesh(core_axis_name="c", subcore_axis_name="s",
                                  num_cores=NSC, num_subcores=NTEC)


def _writeout_split(copy_fn):
    sid = lax.axis_index("s")

    @pl.when(sid < NTEC - 1)
    def _():
        copy_fn(sid * 640, 640)

    @pl.when(sid == NTEC - 1)
    def _():
        copy_fn((NTEC - 1) * 640, N - (NTEC - 1) * 640)


def _root_kernel(first, x, roots, idxv, rows, sem):
    cid = lax.axis_index("c")
    sid = lax.axis_index("s")

    @pl.when(jnp.logical_and(cid == 0, sid == 0))
    def _():
        pltpu.sync_copy(first, idxv)
        pltpu.async_copy(x.at[idxv], rows, sem).wait()
        pltpu.sync_copy(rows, roots)


def _sc_root_gather(first, x):
    k = pl.kernel(
        _root_kernel,
        out_type=jax.ShapeDtypeStruct((G, F), jnp.float32),
        mesh=_sc_mesh(),
        scratch_types=[
            pltpu.VMEM((G,), jnp.int32),
            pltpu.VMEM((G, F), jnp.float32),
            pltpu.SemaphoreType.DMA,
        ],
    )
    return k(first, x)


def _agg_kernel(p, s0, d0, s1, d1, zrows,
                out, acc, sidx_a, didx_a, sidx_b, didx_b,
                rows_a, rows_b, sem_a, sem_b, semi_a, semi_b):
    cid = lax.axis_index("c")
    sid = lax.axis_index("s")
    pltpu.sync_copy(zrows, acc.at[pl.ds(sid * 640, 640)])
    plsc.subcore_barrier()

    for d in (0, 1):
        s_in = s0 if d == 0 else s1
        d_in = d0 if d == 0 else d1

        @pl.when(cid == d)
        def _():
            pd = p.at[d]
            idx = ((sidx_a, didx_a, rows_a, sem_a, semi_a),
                   (sidx_b, didx_b, rows_b, sem_b, semi_b))

            def fire_idx(j, b):
                si, di, _, _, smi = idx[b]
                pltpu.async_copy(s_in.at[sid].at[j], si, smi)
                pltpu.async_copy(d_in.at[sid].at[j], di, smi)

            def wait_idx(b):
                si, di, _, _, smi = idx[b]
                pltpu.make_async_copy(s_in.at[sid].at[0], si, smi).wait()
                pltpu.make_async_copy(d_in.at[sid].at[0], di, smi).wait()

            def fire_gather(b):
                si, _, rows, sm, _ = idx[b]
                pltpu.async_copy(pd.at[si], rows, sm)

            def drain_scatter(b):
                si, di, rows, sm, _ = idx[b]
                pltpu.make_async_copy(pd.at[si], rows, sm).wait()
                pltpu.sync_copy(rows, acc.at[di], add=True)

            fire_idx(0, 0)
            wait_idx(0)
            fire_gather(0)
            fire_idx(1, 1)

            @pl.loop(0, (NCH - 1) // 2)
            def _(j):
                wait_idx(1)
                fire_gather(1)
                drain_scatter(0)
                fire_idx(2 * j + 2, 0)
                wait_idx(0)
                fire_gather(0)
                drain_scatter(1)

                @pl.when(j < (NCH - 1) // 2 - 1)
                def _():
                    fire_idx(2 * j + 3, 1)

            drain_scatter(0)

    plsc.subcore_barrier()

    for d in (0, 1):
        @pl.when(cid == d)
        def _():
            _writeout_split(
                lambda r0, n: pltpu.sync_copy(acc.at[pl.ds(r0, n)],
                                              out.at[d].at[pl.ds(r0, n)]))


def _sc_edge_agg(p, s0, d0, s1, d1, zrows):
    k = pl.kernel(
        _agg_kernel,
        out_type=jax.ShapeDtypeStruct((2, N, F), jnp.float32),
        mesh=_sc_mesh(),
        scratch_types=[
            pltpu.VMEM_SHARED((NPAD, F), jnp.float32),
            pltpu.VMEM((ECH,), jnp.int32),
            pltpu.VMEM((ECH,), jnp.int32),
            pltpu.VMEM((ECH,), jnp.int32),
            pltpu.VMEM((ECH,), jnp.int32),
            pltpu.VMEM((ECH, F), jnp.float32),
            pltpu.VMEM((ECH, F), jnp.float32),
            pltpu.SemaphoreType.DMA,
            pltpu.SemaphoreType.DMA,
            pltpu.SemaphoreType.DMA,
            pltpu.SemaphoreType.DMA,
        ],
    )
    return k(p, s0, d0, s1, d1, zrows)


EB = 512
EBLK = E // EB


def _hist_kernel(e0, e1, d0t, d1t):
    i = pl.program_id(0)

    @pl.when(i == 0)
    def _():
        d0t[...] = jnp.zeros_like(d0t)
        d1t[...] = jnp.zeros_like(d1t)

    iq = lax.broadcasted_iota(jnp.int32, (EB, NBLK), 1)
    ir = lax.broadcasted_iota(jnp.int32, (EB, NB), 1)
    for dst_ref, out in ((e1, d0t), (e0, d1t)):
        dst = dst_ref[0]
        q = dst // NB
        r = dst - q * NB
        ohq = (q == iq).astype(jnp.bfloat16)
        ohr = (r == ir).astype(jnp.bfloat16)
        out[...] += lax.dot_general(ohr, ohq, (((0,), (0,)), ((), ())),
                                    preferred_element_type=jnp.float32)


def _tc_deghist(e0c, e1c):
    return pl.pallas_call(
        _hist_kernel,
        grid=(EBLK,),
        in_specs=[
            pl.BlockSpec((1, EB, 1), lambda i: (i, 0, 0)),
            pl.BlockSpec((1, EB, 1), lambda i: (i, 0, 0)),
        ],
        out_specs=[
            pl.BlockSpec((NB, NBLK), lambda i: (0, 0)),
            pl.BlockSpec((NB, NBLK), lambda i: (0, 0)),
        ],
        out_shape=[
            jax.ShapeDtypeStruct((NB, NBLK), jnp.float32),
            jax.ShapeDtypeStruct((NB, NBLK), jnp.float32),
        ],
    )(e0c, e1c)


def _mm_kernel(x, w, o):
    o[...] = jnp.dot(x[...], w[...], preferred_element_type=jnp.float32)


def _tc_h(x, wcat):
    return pl.pallas_call(
        _mm_kernel,
        grid=(NBLK,),
        in_specs=[
            pl.BlockSpec((NB, F), lambda i: (i, 0)),
            pl.BlockSpec((F, 2 * F), lambda i: (0, 0)),
        ],
        out_specs=pl.BlockSpec((NB, 2 * F), lambda i: (i, 0)),
        out_shape=jax.ShapeDtypeStruct((N, 2 * F), jnp.float32),
    )(x, wcat)


def _scale_kernel(h, degt, p, dis):
    i = pl.program_id(1)
    lane = lax.broadcasted_iota(jnp.int32, (NB, NBLK), 1)
    deg = jnp.sum(degt[0] * (lane == i).astype(jnp.float32),
                  axis=1, keepdims=True)
    d = jax.lax.rsqrt(deg + 1.0)
    p[0] = h[...] * d
    dis[0] = d


def _tc_scale(h, degt):
    return pl.pallas_call(
        _scale_kernel,
        grid=(2, NBLK),
        in_specs=[
            pl.BlockSpec((NB, F), lambda d, i: (i, d)),
            pl.BlockSpec((1, NB, NBLK), lambda d, i: (d, 0, 0)),
        ],
        out_specs=[
            pl.BlockSpec((1, NB, F), lambda d, i: (d, i, 0)),
            pl.BlockSpec((1, NB, 1), lambda d, i: (d, i, 0)),
        ],
        out_shape=[
            jax.ShapeDtypeStruct((2, N, F), jnp.float32),
            jax.ShapeDtypeStruct((2, N, 1), jnp.float32),
        ],
    )(h, degt)


def _first_kernel(bc, first, cnt, cacc):
    i = pl.program_id(0)

    @pl.when(i == 0)
    def _():
        cacc[...] = jnp.zeros_like(cacc)

    gidx = lax.broadcasted_iota(jnp.int32, (NB, G), 1)
    oh = (bc[...] == gidx).astype(jnp.float32)
    cacc[...] += jnp.sum(oh, axis=0, keepdims=True)

    @pl.when(i == NBLK - 1)
    def _():
        c = cacc[...]
        gj = lax.broadcasted_iota(jnp.int32, (G, G), 0)
        gg = lax.broadcasted_iota(jnp.int32, (G, G), 1)
        lt = (gj < gg).astype(jnp.float32)
        f = jnp.dot(c, lt, preferred_element_type=jnp.float32)
        first[...] = jnp.clip(f.astype(jnp.int32), 0, N - 1)
        cnt[...] = c


def _tc_first(batch_c):
    return pl.pallas_call(
        _first_kernel,
        grid=(NBLK,),
        in_specs=[pl.BlockSpec((NB, 1), lambda i: (i, 0))],
        out_specs=[
            pl.BlockSpec((1, G), lambda i: (0, 0)),
            pl.BlockSpec((1, G), lambda i: (0, 0)),
        ],
        out_shape=[
            jax.ShapeDtypeStruct((1, G), jnp.int32),
            jax.ShapeDtypeStruct((1, G), jnp.float32),
        ],
        scratch_shapes=[pltpu.VMEM((1, G), jnp.float32)],
    )(batch_c)


def _q_kernel(roots, wbot, q):
    q[...] = jnp.dot(jax.nn.relu(roots[...]), wbot[...],
                     preferred_element_type=jnp.float32)


def _tc_q(roots, wbot_cat):
    return pl.pallas_call(
        _q_kernel,
        out_shape=jax.ShapeDtypeStruct((G, 2 * F), jnp.float32),
    )(roots, wbot_cat)


def _layer2_kernel(pp, agg, dis, bc, q, wtop, b1, b2, p2, basev):
    b = bc[...]
    gidx = lax.broadcasted_iota(jnp.int32, (NB, G), 1)
    oh = (b == gidx).astype(jnp.float32)
    for d in (0, 1):
        dd = dis[d]
        conv1 = dd * (agg[d] + pp[d]) + b1[d]
        t = jax.nn.relu(conv1)
        h2 = (jnp.dot(t, wtop[d], preferred_element_type=jnp.float32)
              + jnp.dot(oh, q[:, d * F:(d + 1) * F],
                        preferred_element_type=jnp.float32))
        p2d = dd * h2
        p2[d] = p2d
        basev[d] = dd * p2d + b2[d]


def _tc_layer2(p, agg, dis, batch_c, q, wtop, b1, b2):
    return pl.pallas_call(
        _layer2_kernel,
        grid=(NBLK,),
        in_specs=[
            pl.BlockSpec((2, NB, F), lambda i: (0, i, 0)),
            pl.BlockSpec((2, NB, F), lambda i: (0, i, 0)),
            pl.BlockSpec((2, NB, 1), lambda i: (0, i, 0)),
            pl.BlockSpec((NB, 1), lambda i: (i, 0)),
            pl.BlockSpec((G, 2 * F), lambda i: (0, 0)),
            pl.BlockSpec((2, F, F), lambda i: (0, 0, 0)),
            pl.BlockSpec((2, F), lambda i: (0, 0)),
            pl.BlockSpec((2, F), lambda i: (0, 0)),
        ],
        out_specs=[
            pl.BlockSpec((2, NB, F), lambda i: (0, i, 0)),
            pl.BlockSpec((2, NB, F), lambda i: (0, i, 0)),
        ],
        out_shape=[
            jax.ShapeDtypeStruct((2, N, F), jnp.float32),
            jax.ShapeDtypeStruct((2, N, F), jnp.float32),
        ],
    )(p, agg, dis, batch_c, q, wtop, b1, b2)


def _final_kernel(agg2, basev, dis, bc, cnt, lw, lb, out, acc):
    i = pl.program_id(0)

    @pl.when(i == 0)
    def _():
        acc[...] = jnp.zeros_like(acc)

    b = bc[...]
    gidx = lax.broadcasted_iota(jnp.int32, (NB, G), 1)
    oh = (b == gidx).astype(jnp.float32)
    for d in (0, 1):
        h = jax.nn.relu(dis[d] * agg2[d] + basev[d])
        contrib = lax.dot_general(oh, h, (((0,), (0,)), ((), ())),
                                  preferred_element_type=jnp.float32)
        acc[:, d * F:(d + 1) * F] += contrib

    @pl.when(i == NBLK - 1)
    def _():
        gr = lax.broadcasted_iota(jnp.int32, (G, G), 0)
        gc = lax.broadcasted_iota(jnp.int32, (G, G), 1)
        dm = (gr == gc).astype(jnp.float32) / jnp.maximum(cnt[...], 1.0)
        mean = jnp.dot(dm, acc[...], preferred_element_type=jnp.float32)
        logits = jnp.dot(mean, lw[...],
                         preferred_element_type=jnp.float32) + lb[...]
        m = jnp.max(logits, axis=-1, keepdims=True)
        z = logits - m
        out[...] = z - jnp.log(jnp.sum(jnp.exp(z), axis=-1, keepdims=True))


def _tc_final(agg2, basev, dis, batch_c, cnt, lw, lb):
    return pl.pallas_call(
        _final_kernel,
        grid=(NBLK,),
        in_specs=[
            pl.BlockSpec((2, NB, F), lambda i: (0, i, 0)),
            pl.BlockSpec((2, NB, F), lambda i: (0, i, 0)),
            pl.BlockSpec((2, NB, 1), lambda i: (0, i, 0)),
            pl.BlockSpec((NB, 1), lambda i: (i, 0)),
            pl.BlockSpec((1, G), lambda i: (0, 0)),
            pl.BlockSpec((2 * F, C), lambda i: (0, 0)),
            pl.BlockSpec((1, C), lambda i: (0, 0)),
        ],
        out_specs=pl.BlockSpec((G, C), lambda i: (0, 0)),
        out_shape=jax.ShapeDtypeStruct((G, C), jnp.float32),
        scratch_shapes=[pltpu.VMEM((G, 2 * F), jnp.float32)],
    )(agg2, basev, dis, batch_c, cnt, lw, lb)


def kernel(x, edge_index, batch, bu1_W, bu1_b, td1_W, td1_b, root_W, root_b,
           bu2_W, bu2_b, td2_W, td2_b, lin_W, lin_b):
    ei = edge_index.astype(jnp.int32)
    ei0, ei1 = ei[0], ei[1]
    batch_c = batch.astype(jnp.int32).reshape(N, 1)
    zrows = jnp.zeros((640, F), jnp.float32)

    wcat = jnp.concatenate([bu1_W, td1_W], axis=1)
    wtop = jnp.stack([bu2_W[:F], td2_W[:F]])
    wbot_cat = jnp.concatenate([bu2_W[F:], td2_W[F:]], axis=1)
    b1 = jnp.stack([bu1_b, td1_b])
    b2 = jnp.stack([bu2_b, td2_b])

    e0c = ei0.reshape(EBLK, EB, 1)
    e1c = ei1.reshape(EBLK, EB, 1)
    d0t, d1t = _tc_deghist(e0c, e1c)
    h = _tc_h(x, wcat)

    p, dis = _tc_scale(h, jnp.stack([d0t, d1t]))
    first, cnt = _tc_first(batch_c)

    roots = _sc_root_gather(first.reshape(G), x)
    q = _tc_q(roots, wbot_cat)

    def chunked(e, pad):
        return jnp.concatenate(
            [e, jnp.full((EPAD - E,), pad, jnp.int32)]).reshape(NTEC, NCH, ECH)

    s0 = chunked(ei0, 0)
    d0 = chunked(ei1, N)
    s1 = chunked(ei1, 0)
    d1 = chunked(ei0, N)

    agg = _sc_edge_agg(p, s0, d0, s1, d1, zrows)
    p2, basev = _tc_layer2(p, agg, dis, batch_c, q, wtop, b1, b2)
    agg2 = _sc_edge_agg(p2, s0, d0, s1, d1, zrows)
    return _tc_final(agg2, basev, dis, batch_c, cnt,
                     lin_W, lin_b.reshape(1, C))


# speedup vs baseline: 20.6798x; 2.2487x over previous
import functools

import jax
import jax.numpy as jnp
from jax import lax
from jax.experimental import pallas as pl
from jax.experimental.pallas import tpu as pltpu
from jax.experimental.pallas import tpu_sc as plsc

N = 10000
E = 320000
F = 128
G = 64
C = 4
NB = 400
NBLK = N // NB
NSC = 2
NTEC = 16
ECH = 128
NCH = 157
EPAD = NTEC * NCH * ECH
NPAD = 10240


def _sc_mesh():
    return plsc.VectorSubcoreMesh(core_axis_name="c", subcore_axis_name="s",
                                  num_cores=NSC, num_subcores=NTEC)


def _writeout_split(copy_fn):
    sid = lax.axis_index("s")

    @pl.when(sid < NTEC - 1)
    def _():
        copy_fn(sid * 640, 640)

    @pl.when(sid == NTEC - 1)
    def _():
        copy_fn((NTEC - 1) * 640, N - (NTEC - 1) * 640)


def _root_kernel(first, x, roots, idxv, rows, sem):
    cid = lax.axis_index("c")
    sid = lax.axis_index("s")

    @pl.when(jnp.logical_and(cid == 0, sid == 0))
    def _():
        pltpu.sync_copy(first, idxv)
        pltpu.async_copy(x.at[idxv], rows, sem).wait()
        pltpu.sync_copy(rows, roots)


def _sc_root_gather(first, x):
    k = pl.kernel(
        _root_kernel,
        out_type=jax.ShapeDtypeStruct((G, F), jnp.float32),
        mesh=_sc_mesh(),
        scratch_types=[
            pltpu.VMEM((G,), jnp.int32),
            pltpu.VMEM((G, F), jnp.float32),
            pltpu.SemaphoreType.DMA,
        ],
    )
    return k(first, x)


def _agg_kernel(p, s0, d0, s1, d1, zrows,
                out, acc, sidx_a, didx_a, sidx_b, didx_b,
                rows_a, rows_b, sem_a, sem_b, semi_a, semi_b):
    cid = lax.axis_index("c")
    sid = lax.axis_index("s")
    pltpu.sync_copy(zrows, acc.at[pl.ds(sid * 640, 640)])
    plsc.subcore_barrier()

    for d in (0, 1):
        s_in = s0 if d == 0 else s1
        d_in = d0 if d == 0 else d1

        @pl.when(cid == d)
        def _():
            pd = p.at[d]
            idx = ((sidx_a, didx_a, rows_a, sem_a, semi_a),
                   (sidx_b, didx_b, rows_b, sem_b, semi_b))

            def fire_idx(j, b):
                si, di, _, _, smi = idx[b]
                off = (sid * NCH + j) * ECH
                pltpu.async_copy(s_in.at[pl.ds(off, ECH)], si, smi)
                pltpu.async_copy(d_in.at[pl.ds(off, ECH)], di, smi)

            def wait_idx(b):
                si, di, _, _, smi = idx[b]
                pltpu.make_async_copy(s_in.at[pl.ds(0, ECH)], si, smi).wait()
                pltpu.make_async_copy(d_in.at[pl.ds(0, ECH)], di, smi).wait()

            def fire_gather(b):
                si, _, rows, sm, _ = idx[b]
                pltpu.async_copy(pd.at[si], rows, sm)

            def drain_scatter(b):
                si, di, rows, sm, _ = idx[b]
                pltpu.make_async_copy(pd.at[si], rows, sm).wait()
                pltpu.sync_copy(rows, acc.at[di], add=True)

            fire_idx(0, 0)
            wait_idx(0)
            fire_gather(0)
            fire_idx(1, 1)

            @pl.loop(0, (NCH - 1) // 2)
            def _(j):
                wait_idx(1)
                fire_gather(1)
                drain_scatter(0)
                fire_idx(2 * j + 2, 0)
                wait_idx(0)
                fire_gather(0)
                drain_scatter(1)

                @pl.when(j < (NCH - 1) // 2 - 1)
                def _():
                    fire_idx(2 * j + 3, 1)

            drain_scatter(0)

    plsc.subcore_barrier()

    for d in (0, 1):
        @pl.when(cid == d)
        def _():
            _writeout_split(
                lambda r0, n: pltpu.sync_copy(acc.at[pl.ds(r0, n)],
                                              out.at[d].at[pl.ds(r0, n)]))


def _sc_edge_agg(p, s0, d0, s1, d1, zrows):
    k = pl.kernel(
        _agg_kernel,
        out_type=jax.ShapeDtypeStruct((2, N, F), jnp.float32),
        mesh=_sc_mesh(),
        scratch_types=[
            pltpu.VMEM_SHARED((NPAD, F), jnp.float32),
            pltpu.VMEM((ECH,), jnp.int32),
            pltpu.VMEM((ECH,), jnp.int32),
            pltpu.VMEM((ECH,), jnp.int32),
            pltpu.VMEM((ECH,), jnp.int32),
            pltpu.VMEM((ECH, F), jnp.float32),
            pltpu.VMEM((ECH, F), jnp.float32),
            pltpu.SemaphoreType.DMA,
            pltpu.SemaphoreType.DMA,
            pltpu.SemaphoreType.DMA,
            pltpu.SemaphoreType.DMA,
        ],
    )
    return k(p, s0, d0, s1, d1, zrows)


EB = 3200
EBLK = E // EB
DD = 100


def _hist_kernel(ei, dh):
    i = pl.program_id(0)

    @pl.when(i == 0)
    def _():
        dh[...] = jnp.zeros_like(dh)

    ia = lax.broadcasted_iota(jnp.int32, (DD, EB), 0)
    for d in (0, 1):
        dst = ei[1 - d:2 - d, :]
        a = dst // DD
        b = dst - a * DD
        oha = (a == ia).astype(jnp.bfloat16)
        ohb = (b == ia).astype(jnp.bfloat16)
        dh[d] += lax.dot_general(ohb, oha, (((1,), (1,)), ((), ())),
                                 preferred_element_type=jnp.float32)


def _tc_deghist(ei):
    return pl.pallas_call(
        _hist_kernel,
        grid=(EBLK,),
        in_specs=[pl.BlockSpec((2, EB), lambda i: (0, i))],
        out_specs=pl.BlockSpec((2, DD, DD), lambda i: (0, 0, 0)),
        out_shape=jax.ShapeDtypeStruct((2, DD, DD), jnp.float32),
    )(ei)


def _prep_kernel(ei, s0, d0, s1, d1):
    e0 = ei[0:1, :]
    e1 = ei[1:2, :]
    padz = jnp.zeros((1, EPAD - E), jnp.int32)
    padn = jnp.full((1, EPAD - E), N, jnp.int32)
    s0[...] = jnp.concatenate([e0, padz], axis=1).reshape(EPAD)
    d0[...] = jnp.concatenate([e1, padn], axis=1).reshape(EPAD)
    s1[...] = jnp.concatenate([e1, padz], axis=1).reshape(EPAD)
    d1[...] = jnp.concatenate([e0, padn], axis=1).reshape(EPAD)


def _tc_edgeprep(ei):
    return pl.pallas_call(
        _prep_kernel,
        out_shape=[jax.ShapeDtypeStruct((EPAD,), jnp.int32)] * 4,
    )(ei)


def _mm_kernel(x, w, o):
    o[...] = jnp.dot(x[...], w[...], preferred_element_type=jnp.float32)


def _tc_h(x, wcat):
    return pl.pallas_call(
        _mm_kernel,
        grid=(NBLK,),
        in_specs=[
            pl.BlockSpec((NB, F), lambda i: (i, 0)),
            pl.BlockSpec((F, 2 * F), lambda i: (0, 0)),
        ],
        out_specs=pl.BlockSpec((NB, 2 * F), lambda i: (i, 0)),
        out_shape=jax.ShapeDtypeStruct((N, 2 * F), jnp.float32),
    )(x, wcat)


def _scale_kernel(h, dh, p, dis):
    i = pl.program_id(1)
    dmat = dh[0]
    lane = lax.broadcasted_iota(jnp.int32, (DD, DD), 1)
    cols = []
    for k in range(NB // DD):
        sel = (lane == (NB // DD) * i + k).astype(jnp.float32)
        cols.append(jnp.sum(dmat * sel, axis=1, keepdims=True))
    deg = jnp.concatenate(cols, axis=0)
    d = jax.lax.rsqrt(deg + 1.0)
    p[0] = h[...] * d
    dis[0] = d


def _tc_scale(h, dh):
    return pl.pallas_call(
        _scale_kernel,
        grid=(2, NBLK),
        in_specs=[
            pl.BlockSpec((NB, F), lambda d, i: (i, d)),
            pl.BlockSpec((1, DD, DD), lambda d, i: (d, 0, 0)),
        ],
        out_specs=[
            pl.BlockSpec((1, NB, F), lambda d, i: (d, i, 0)),
            pl.BlockSpec((1, NB, 1), lambda d, i: (d, i, 0)),
        ],
        out_shape=[
            jax.ShapeDtypeStruct((2, N, F), jnp.float32),
            jax.ShapeDtypeStruct((2, N, 1), jnp.float32),
        ],
    )(h, dh)


def _first_kernel(bc, first, cnt, cacc):
    i = pl.program_id(0)

    @pl.when(i == 0)
    def _():
        cacc[...] = jnp.zeros_like(cacc)

    gidx = lax.broadcasted_iota(jnp.int32, (NB, G), 1)
    oh = (bc[...] == gidx).astype(jnp.float32)
    cacc[...] += jnp.sum(oh, axis=0, keepdims=True)

    @pl.when(i == NBLK - 1)
    def _():
        c = cacc[...]
        gj = lax.broadcasted_iota(jnp.int32, (G, G), 0)
        gg = lax.broadcasted_iota(jnp.int32, (G, G), 1)
        lt = (gj < gg).astype(jnp.float32)
        f = jnp.dot(c, lt, preferred_element_type=jnp.float32)
        first[...] = jnp.clip(f.astype(jnp.int32), 0, N - 1)
        cnt[...] = c


def _tc_first(batch_c):
    return pl.pallas_call(
        _first_kernel,
        grid=(NBLK,),
        in_specs=[pl.BlockSpec((NB, 1), lambda i: (i, 0))],
        out_specs=[
            pl.BlockSpec((1, G), lambda i: (0, 0)),
            pl.BlockSpec((1, G), lambda i: (0, 0)),
        ],
        out_shape=[
            jax.ShapeDtypeStruct((1, G), jnp.int32),
            jax.ShapeDtypeStruct((1, G), jnp.float32),
        ],
        scratch_shapes=[pltpu.VMEM((1, G), jnp.float32)],
    )(batch_c)


def _q_kernel(roots, wbot, q):
    q[...] = jnp.dot(jax.nn.relu(roots[...]), wbot[...],
                     preferred_element_type=jnp.float32)


def _tc_q(roots, wbot_cat):
    return pl.pallas_call(
        _q_kernel,
        out_shape=jax.ShapeDtypeStruct((G, 2 * F), jnp.float32),
    )(roots, wbot_cat)


def _layer2_kernel(pp, agg, dis, bc, q, wtop, b1, b2, p2, basev):
    b = bc[...]
    gidx = lax.broadcasted_iota(jnp.int32, (NB, G), 1)
    oh = (b == gidx).astype(jnp.float32)
    for d in (0, 1):
        dd = dis[d]
        conv1 = dd * (agg[d] + pp[d]) + b1[d]
        t = jax.nn.relu(conv1)
        h2 = (jnp.dot(t, wtop[d], preferred_element_type=jnp.float32)
              + jnp.dot(oh, q[:, d * F:(d + 1) * F],
                        preferred_element_type=jnp.float32))
        p2d = dd * h2
        p2[d] = p2d
        basev[d] = dd * p2d + b2[d]


def _tc_layer2(p, agg, dis, batch_c, q, wtop, b1, b2):
    return pl.pallas_call(
        _layer2_kernel,
        grid=(NBLK,),
        in_specs=[
            pl.BlockSpec((2, NB, F), lambda i: (0, i, 0)),
            pl.BlockSpec((2, NB, F), lambda i: (0, i, 0)),
            pl.BlockSpec((2, NB, 1), lambda i: (0, i, 0)),
            pl.BlockSpec((NB, 1), lambda i: (i, 0)),
            pl.BlockSpec((G, 2 * F), lambda i: (0, 0)),
            pl.BlockSpec((2, F, F), lambda i: (0, 0, 0)),
            pl.BlockSpec((2, F), lambda i: (0, 0)),
            pl.BlockSpec((2, F), lambda i: (0, 0)),
        ],
        out_specs=[
            pl.BlockSpec((2, NB, F), lambda i: (0, i, 0)),
            pl.BlockSpec((2, NB, F), lambda i: (0, i, 0)),
        ],
        out_shape=[
            jax.ShapeDtypeStruct((2, N, F), jnp.float32),
            jax.ShapeDtypeStruct((2, N, F), jnp.float32),
        ],
    )(p, agg, dis, batch_c, q, wtop, b1, b2)


def _final_kernel(agg2, basev, dis, bc, cnt, lw, lb, out, acc):
    i = pl.program_id(0)

    @pl.when(i == 0)
    def _():
        acc[...] = jnp.zeros_like(acc)

    b = bc[...]
    gidx = lax.broadcasted_iota(jnp.int32, (NB, G), 1)
    oh = (b == gidx).astype(jnp.float32)
    for d in (0, 1):
        h = jax.nn.relu(dis[d] * agg2[d] + basev[d])
        contrib = lax.dot_general(oh, h, (((0,), (0,)), ((), ())),
                                  preferred_element_type=jnp.float32)
        acc[:, d * F:(d + 1) * F] += contrib

    @pl.when(i == NBLK - 1)
    def _():
        gr = lax.broadcasted_iota(jnp.int32, (G, G), 0)
        gc = lax.broadcasted_iota(jnp.int32, (G, G), 1)
        dm = (gr == gc).astype(jnp.float32) / jnp.maximum(cnt[...], 1.0)
        mean = jnp.dot(dm, acc[...], preferred_element_type=jnp.float32)
        logits = jnp.dot(mean, lw[...],
                         preferred_element_type=jnp.float32) + lb[...]
        m = jnp.max(logits, axis=-1, keepdims=True)
        z = logits - m
        out[...] = z - jnp.log(jnp.sum(jnp.exp(z), axis=-1, keepdims=True))


def _tc_final(agg2, basev, dis, batch_c, cnt, lw, lb):
    return pl.pallas_call(
        _final_kernel,
        grid=(NBLK,),
        in_specs=[
            pl.BlockSpec((2, NB, F), lambda i: (0, i, 0)),
            pl.BlockSpec((2, NB, F), lambda i: (0, i, 0)),
            pl.BlockSpec((2, NB, 1), lambda i: (0, i, 0)),
            pl.BlockSpec((NB, 1), lambda i: (i, 0)),
            pl.BlockSpec((1, G), lambda i: (0, 0)),
            pl.BlockSpec((2 * F, C), lambda i: (0, 0)),
            pl.BlockSpec((1, C), lambda i: (0, 0)),
        ],
        out_specs=pl.BlockSpec((G, C), lambda i: (0, 0)),
        out_shape=jax.ShapeDtypeStruct((G, C), jnp.float32),
        scratch_shapes=[pltpu.VMEM((G, 2 * F), jnp.float32)],
    )(agg2, basev, dis, batch_c, cnt, lw, lb)


def kernel(x, edge_index, batch, bu1_W, bu1_b, td1_W, td1_b, root_W, root_b,
           bu2_W, bu2_b, td2_W, td2_b, lin_W, lin_b):
    ei = edge_index.astype(jnp.int32)
    batch_c = batch.astype(jnp.int32).reshape(N, 1)
    zrows = jnp.zeros((640, F), jnp.float32)

    wcat = jnp.concatenate([bu1_W, td1_W], axis=1)
    wtop = jnp.stack([bu2_W[:F], td2_W[:F]])
    wbot_cat = jnp.concatenate([bu2_W[F:], td2_W[F:]], axis=1)
    b1 = jnp.stack([bu1_b, td1_b])
    b2 = jnp.stack([bu2_b, td2_b])

    dh = _tc_deghist(ei)
    h = _tc_h(x, wcat)

    p, dis = _tc_scale(h, dh)
    first, cnt = _tc_first(batch_c)

    roots = _sc_root_gather(first.reshape(G), x)
    q = _tc_q(roots, wbot_cat)

    s0, d0, s1, d1 = _tc_edgeprep(ei)

    agg = _sc_edge_agg(p, s0, d0, s1, d1, zrows)
    p2, basev = _tc_layer2(p, agg, dis, batch_c, q, wtop, b1, b2)
    agg2 = _sc_edge_agg(p2, s0, d0, s1, d1, zrows)
    return _tc_final(agg2, basev, dis, batch_c, cnt,
                     lin_W, lin_b.reshape(1, C))
